# baseline (speedup 1.0000x reference)
import functools

import jax
import jax.numpy as jnp
from jax import lax
from jax.experimental import pallas as pl
from jax.experimental.pallas import tpu as pltpu
from jax.experimental.pallas import tpu_sc as plsc

N = 10000
E = 320000
D = 64
NUM_CLASSES = 47
EPS_BN = 1e-5

NC, NS = 2, 16
NW = NC * NS
CH = 128
NCHUNK = 79
EPW = NCHUNK * CH
EPAD = NW * EPW - E
SINK = N
NACC = N + 16
RPT = 624
TAIL = N - NS * RPT


def _sc_aggregate_call(y, src_r, dst_r):
  mesh = plsc.VectorSubcoreMesh(core_axis_name="c", subcore_axis_name="s")

  @functools.partial(
      pl.kernel,
      out_type=jax.ShapeDtypeStruct((NC, N, D), jnp.float32),
      mesh=mesh,
      scratch_types=[
          pltpu.VMEM((NCHUNK, CH), jnp.int32),
          pltpu.VMEM((NCHUNK, CH), jnp.int32),
          pltpu.VMEM((2, CH, D), jnp.float32),
          pltpu.VMEM_SHARED((NACC, D), jnp.float32),
          pltpu.VMEM_SHARED((N, D), jnp.float32),
          pltpu.SemaphoreType.DMA((2,)),
          pltpu.SemaphoreType.DMA((3,)),
      ],
      compiler_params=pltpu.CompilerParams(use_tc_tiling_on_sc=False),
  )
  def agg(y_hbm, src_hbm, dst_hbm, out_hbm, src_v, dst_v, rows_v,
          acc_s, y_s, sem, ssem):
    cid = lax.axis_index("c")
    sid = lax.axis_index("s")
    wid = sid * NC + cid

    c1 = pltpu.async_copy(y_hbm.at[pl.ds(sid * RPT, RPT)],
                          y_s.at[pl.ds(sid * RPT, RPT)], ssem.at[0])
    c2 = pltpu.async_copy(y_hbm.at[pl.ds(sid * RPT, RPT)],
                          acc_s.at[pl.ds(sid * RPT, RPT)], ssem.at[1])
    c3 = pltpu.async_copy(src_hbm.at[wid], src_v, ssem.at[2])
    pltpu.async_copy(dst_hbm.at[wid], dst_v, ssem.at[2])
    c1.wait()
    c2.wait()
    c3.wait()
    c3.wait()

    @pl.when(sid == NS - 1)
    def _seed_tail():
      pltpu.sync_copy(y_hbm.at[pl.ds(NS * RPT, TAIL)],
                      y_s.at[pl.ds(NS * RPT, TAIL)])
      pltpu.sync_copy(y_hbm.at[pl.ds(NS * RPT, TAIL)],
                      acc_s.at[pl.ds(NS * RPT, TAIL)])

    plsc.subcore_barrier()

    pltpu.async_copy(y_s.at[src_v.at[0]], rows_v.at[0], sem.at[0])

    def step(j, carry):
      b = lax.rem(j, 2)
      pltpu.make_async_copy(y_s.at[src_v.at[j]], rows_v.at[b],
                            sem.at[b]).wait()

      @pl.when(j < NCHUNK - 1)
      def _prefetch():
        nb = 1 - b
        pltpu.async_copy(y_s.at[src_v.at[j + 1]], rows_v.at[nb], sem.at[nb])

      pltpu.sync_copy(rows_v.at[b], acc_s.at[dst_v.at[j]], add=True)
      return carry

    lax.fori_loop(0, NCHUNK, step, 0, unroll=2)

    plsc.subcore_barrier()
    pltpu.sync_copy(acc_s.at[pl.ds(sid * RPT, RPT)],
                    out_hbm.at[cid, pl.ds(sid * RPT, RPT)])

    @pl.when(sid == NS - 1)
    def _write_tail():
      pltpu.sync_copy(acc_s.at[pl.ds(NS * RPT, TAIL)],
                      out_hbm.at[cid, pl.ds(NS * RPT, TAIL)])

  return agg(y, src_r, dst_r)


NP = N // 2
DP = 2 * D


NBLK = 5
BR = NP // NBLK


def _mm_body(x_ref, w_ref, o_ref):
  o_ref[...] = lax.dot_general(
      x_ref[...], w_ref[...], (((1,), (0,)), ((), ())),
      preferred_element_type=jnp.float32, precision=lax.Precision.HIGHEST)


def _input_matmul(x2, b0):
  k = x2.shape[1]
  return pl.pallas_call(
      _mm_body,
      grid=(NBLK,),
      in_specs=[pl.BlockSpec((BR, k), lambda i: (i, 0)),
                pl.BlockSpec((k, DP), lambda i: (0, 0))],
      out_specs=pl.BlockSpec((BR, DP), lambda i: (i, 0)),
      out_shape=jax.ShapeDtypeStruct((NP, DP), jnp.float32),
  )(x2, b0)


def _layer_body(y_ref, acc_ref, b_ref, g_ref, be_ref, p_ref, w_ref, o_ref,
                z_sc, s_sc, v_sc):
  ph = pl.program_id(0)
  blk = pl.program_id(1)

  fold = lambda s: lax.dot_general(
      s, p_ref[...], (((1,), (0,)), ((), ())),
      preferred_element_type=jnp.float32,
      precision=lax.Precision.HIGHEST) * (1.0 / N)

  @pl.when(ph == 0)
  def _accumulate():
    z = acc_ref[0] + acc_ref[1] - y_ref[...] + b_ref[...]
    z_sc[blk] = z

    @pl.when(blk == 0)
    def _init():
      s_sc[...] = jnp.zeros_like(s_sc)

    s_sc[...] += jnp.sum(z, axis=0, keepdims=True)

  @pl.when(ph == 1)
  def _variance():
    d = z_sc[blk] - fold(s_sc[...])

    @pl.when(blk == 0)
    def _init():
      v_sc[...] = jnp.zeros_like(v_sc)

    v_sc[...] += jnp.sum(d * d, axis=0, keepdims=True)

  @pl.when(ph == 2)
  def _apply():
    d = z_sc[blk] - fold(s_sc[...])
    h = d * lax.rsqrt(fold(v_sc[...]) + EPS_BN) * g_ref[...] + be_ref[...]
    h = jnp.maximum(h, 0.0)
    o_ref[...] = lax.dot_general(
        h, w_ref[...], (((1,), (0,)), ((), ())),
        preferred_element_type=jnp.float32, precision=lax.Precision.HIGHEST)


def _tc_layer(y2, acc2, b, g, be, p, w_next):
  row_p0 = lambda ph, blk: (jnp.where(ph == 0, blk, 0), 0)
  row = lambda ph, blk: (jnp.where(ph == 2, blk, 0), 0)
  full = lambda ph, blk: (0, 0)
  return pl.pallas_call(
      _layer_body,
      grid=(3, NBLK),
      in_specs=[pl.BlockSpec((BR, DP), row_p0),
                pl.BlockSpec((2, BR, DP),
                             lambda ph, blk: (0, jnp.where(ph == 0, blk, 0), 0)),
                pl.BlockSpec((1, DP), full),
                pl.BlockSpec((1, DP), full),
                pl.BlockSpec((1, DP), full),
                pl.BlockSpec((DP, DP), full),
                pl.BlockSpec((DP, DP), full)],
      out_specs=pl.BlockSpec((BR, DP), row),
      out_shape=jax.ShapeDtypeStruct((NP, DP), jnp.float32),
      scratch_shapes=[pltpu.VMEM((NBLK, BR, DP), jnp.float32),
                      pltpu.VMEM((1, DP), jnp.float32),
                      pltpu.VMEM((1, DP), jnp.float32)],
  )(y2, acc2, b, g, be, p, w_next)


def _final_body(y_ref, acc_ref, b_ref, o_ref):
  o_ref[...] = acc_ref[0] + acc_ref[1] - y_ref[...] + b_ref[...]


def _tc_final(y2, acc2, b):
  return pl.pallas_call(
      _final_body,
      grid=(NBLK,),
      in_specs=[pl.BlockSpec((BR, DP), lambda i: (i, 0)),
                pl.BlockSpec((2, BR, DP), lambda i: (0, i, 0)),
                pl.BlockSpec((1, DP), lambda i: (0, 0))],
      out_specs=pl.BlockSpec((BR, DP), lambda i: (i, 0)),
      out_shape=jax.ShapeDtypeStruct((NP, DP), jnp.float32),
  )(y2, acc2, b)


def _blockdiag2(m):
  a, b = m.shape
  zero = jnp.zeros((a, b), jnp.float32)
  return jnp.concatenate(
      [jnp.concatenate([m, zero], axis=1),
       jnp.concatenate([zero, m], axis=1)], axis=0)


def kernel(x, edge_index, Ws, bs, gammas, betas):
  src_r = jnp.concatenate(
      [edge_index[0], jnp.zeros((EPAD,), jnp.int32)]).reshape(NW, NCHUNK, CH)
  dst_r = jnp.concatenate(
      [edge_index[1], jnp.full((EPAD,), SINK, jnp.int32)]).reshape(
          NW, NCHUNK, CH)

  pad = D - NUM_CLASSES
  w4 = jnp.concatenate([Ws[4], jnp.zeros((pad, D), jnp.float32)], axis=0)
  b4 = jnp.concatenate([bs[4], jnp.zeros((pad,), jnp.float32)])
  b4_2 = jnp.tile(b4, 2).reshape(1, DP)
  bd_next = [_blockdiag2(Ws[i].T) for i in range(1, 4)] + [_blockdiag2(w4.T)]
  proj = jnp.tile(jnp.eye(D, dtype=jnp.float32), (2, 2))

  x2 = x.reshape(NP, 2 * x.shape[1])
  y2 = _input_matmul(x2, _blockdiag2(Ws[0].T))
  for i in range(4):
    acc = _sc_aggregate_call(y2.reshape(N, D), src_r, dst_r)
    y2 = _tc_layer(y2, acc.reshape(NC, NP, DP), jnp.tile(bs[i], 2).reshape(1, DP),
                   jnp.tile(gammas[i], 2).reshape(1, DP),
                   jnp.tile(betas[i], 2).reshape(1, DP), proj, bd_next[i])
  acc = _sc_aggregate_call(y2.reshape(N, D), src_r, dst_r)
  z2 = _tc_final(y2, acc.reshape(NC, NP, DP), b4_2)
  return z2.reshape(N, D)[:, :NUM_CLASSES]


# speedup vs baseline: 1.0635x; 1.0635x over previous
import functools

import jax
import jax.numpy as jnp
from jax import lax
from jax.experimental import pallas as pl
from jax.experimental.pallas import tpu as pltpu
from jax.experimental.pallas import tpu_sc as plsc

N = 10000
E = 320000
D = 64
NUM_CLASSES = 47
EPS_BN = 1e-5

NC, NS = 2, 16
NW = NC * NS
CH = 128
NROWS = E // CH
RPW = NROWS // NW
NEXTRA = NROWS - NW * RPW
NACC = N + 16
RPT = 624
TAIL = N - NS * RPT


def _sc_aggregate_call(y, e_r):
  mesh = plsc.VectorSubcoreMesh(core_axis_name="c", subcore_axis_name="s")

  @functools.partial(
      pl.kernel,
      out_type=jax.ShapeDtypeStruct((NC, N, D), jnp.float32),
      mesh=mesh,
      scratch_types=[
          pltpu.VMEM((RPW + 1, CH), jnp.int32),
          pltpu.VMEM((RPW + 1, CH), jnp.int32),
          pltpu.VMEM((2, CH, D), jnp.float32),
          pltpu.VMEM_SHARED((NACC, D), jnp.float32),
          pltpu.VMEM_SHARED((N, D), jnp.float32),
          pltpu.SemaphoreType.DMA((2,)),
          pltpu.SemaphoreType.DMA((3,)),
      ],
      compiler_params=pltpu.CompilerParams(use_tc_tiling_on_sc=False),
  )
  def agg(y_hbm, e_hbm, out_hbm, src_v, dst_v, rows_v,
          acc_s, y_s, sem, ssem):
    cid = lax.axis_index("c")
    sid = lax.axis_index("s")
    wid = sid * NC + cid
    nch = jnp.where(wid < NEXTRA, RPW + 1, RPW)

    c1 = pltpu.async_copy(y_hbm.at[pl.ds(sid * RPT, RPT)],
                          y_s.at[pl.ds(sid * RPT, RPT)], ssem.at[0])
    c2 = pltpu.async_copy(y_hbm.at[pl.ds(sid * RPT, RPT)],
                          acc_s.at[pl.ds(sid * RPT, RPT)], ssem.at[1])
    c3 = pltpu.async_copy(e_hbm.at[0, pl.ds(wid * RPW, RPW)],
                          src_v.at[pl.ds(0, RPW)], ssem.at[2])
    pltpu.async_copy(e_hbm.at[1, pl.ds(wid * RPW, RPW)],
                     dst_v.at[pl.ds(0, RPW)], ssem.at[2])
    c1.wait()
    c2.wait()
    c3.wait()
    c3.wait()

    @pl.when(wid < NEXTRA)
    def _stage_extra_row():
      pltpu.sync_copy(e_hbm.at[0, pl.ds(NW * RPW + wid, 1)],
                      src_v.at[pl.ds(RPW, 1)])
      pltpu.sync_copy(e_hbm.at[1, pl.ds(NW * RPW + wid, 1)],
                      dst_v.at[pl.ds(RPW, 1)])

    @pl.when(sid == NS - 1)
    def _seed_tail():
      pltpu.sync_copy(y_hbm.at[pl.ds(NS * RPT, TAIL)],
                      y_s.at[pl.ds(NS * RPT, TAIL)])
      pltpu.sync_copy(y_hbm.at[pl.ds(NS * RPT, TAIL)],
                      acc_s.at[pl.ds(NS * RPT, TAIL)])

    plsc.subcore_barrier()

    pltpu.async_copy(y_s.at[src_v.at[0]], rows_v.at[0], sem.at[0])

    def step(j, carry):
      b = lax.rem(j, 2)
      pltpu.make_async_copy(y_s.at[src_v.at[j]], rows_v.at[b],
                            sem.at[b]).wait()

      @pl.when(j < nch - 1)
      def _prefetch():
        nb = 1 - b
        pltpu.async_copy(y_s.at[src_v.at[j + 1]], rows_v.at[nb], sem.at[nb])

      pltpu.sync_copy(rows_v.at[b], acc_s.at[dst_v.at[j]], add=True)
      return carry

    lax.fori_loop(0, RPW, step, 0, unroll=2)

    @pl.when(wid < NEXTRA)
    def _extra_chunk():
      eb = RPW % 2
      pltpu.make_async_copy(y_s.at[src_v.at[RPW]], rows_v.at[eb],
                            sem.at[eb]).wait()
      pltpu.sync_copy(rows_v.at[eb], acc_s.at[dst_v.at[RPW]], add=True)

    plsc.subcore_barrier()
    pltpu.sync_copy(acc_s.at[pl.ds(sid * RPT, RPT)],
                    out_hbm.at[cid, pl.ds(sid * RPT, RPT)])

    @pl.when(sid == NS - 1)
    def _write_tail():
      pltpu.sync_copy(acc_s.at[pl.ds(NS * RPT, TAIL)],
                      out_hbm.at[cid, pl.ds(NS * RPT, TAIL)])

  return agg(y, e_r)


NP = N // 2
DP = 2 * D


def _mm_body(x_ref, w_ref, o_ref):
  o_ref[...] = lax.dot_general(
      x_ref[...], w_ref[...], (((1,), (0,)), ((), ())),
      preferred_element_type=jnp.float32, precision=lax.Precision.HIGHEST)


def _input_matmul(x2, b0):
  return pl.pallas_call(
      _mm_body,
      out_shape=jax.ShapeDtypeStruct((NP, DP), jnp.float32),
  )(x2, b0)


def _layer_body(y_ref, acc_ref, b_ref, g_ref, be_ref, p_ref, w_ref, o_ref):
  z = acc_ref[0] + acc_ref[1] - y_ref[...] + b_ref[...]
  fold = lambda s: lax.dot_general(
      s, p_ref[...], (((1,), (0,)), ((), ())),
      preferred_element_type=jnp.float32,
      precision=lax.Precision.HIGHEST) * (1.0 / N)
  mean = fold(jnp.sum(z, axis=0, keepdims=True))
  d = z - mean
  var = fold(jnp.sum(d * d, axis=0, keepdims=True))
  h = d * lax.rsqrt(var + EPS_BN) * g_ref[...] + be_ref[...]
  h = jnp.maximum(h, 0.0)
  o_ref[...] = lax.dot_general(
      h, w_ref[...], (((1,), (0,)), ((), ())),
      preferred_element_type=jnp.float32, precision=lax.Precision.HIGHEST)


def _tc_layer(y2, acc2, b, g, be, p, w_next):
  return pl.pallas_call(
      _layer_body,
      out_shape=jax.ShapeDtypeStruct((NP, DP), jnp.float32),
  )(y2, acc2, b, g, be, p, w_next)


def _final_body(y_ref, acc_ref, b_ref, o_ref):
  o_ref[...] = acc_ref[0] + acc_ref[1] - y_ref[...] + b_ref[...]


def _tc_final(y2, acc2, b):
  return pl.pallas_call(
      _final_body,
      out_shape=jax.ShapeDtypeStruct((NP, DP), jnp.float32),
  )(y2, acc2, b)


def _blockdiag2(m):
  a, b = m.shape
  zero = jnp.zeros((a, b), jnp.float32)
  return jnp.concatenate(
      [jnp.concatenate([m, zero], axis=1),
       jnp.concatenate([zero, m], axis=1)], axis=0)


def kernel(x, edge_index, Ws, bs, gammas, betas):
  e_r = edge_index.reshape(2, NROWS, CH)

  pad = D - NUM_CLASSES
  w4 = jnp.concatenate([Ws[4], jnp.zeros((pad, D), jnp.float32)], axis=0)
  b4 = jnp.concatenate([bs[4], jnp.zeros((pad,), jnp.float32)])
  b4_2 = jnp.tile(b4, 2).reshape(1, DP)
  bd_next = [_blockdiag2(Ws[i].T) for i in range(1, 4)] + [_blockdiag2(w4.T)]
  proj = jnp.tile(jnp.eye(D, dtype=jnp.float32), (2, 2))

  x2 = x.reshape(NP, 2 * x.shape[1])
  y2 = _input_matmul(x2, _blockdiag2(Ws[0].T))
  for i in range(4):
    acc = _sc_aggregate_call(y2.reshape(N, D), e_r)
    y2 = _tc_layer(y2, acc.reshape(NC, NP, DP), jnp.tile(bs[i], 2).reshape(1, DP),
                   jnp.tile(gammas[i], 2).reshape(1, DP),
                   jnp.tile(betas[i], 2).reshape(1, DP), proj, bd_next[i])
  acc = _sc_aggregate_call(y2.reshape(N, D), e_r)
  z2 = _tc_final(y2, acc.reshape(NC, NP, DP), b4_2)
  return z2.reshape(N, D)[:, :NUM_CLASSES]
